# baseline (speedup 1.0000x reference)
import math
from functools import partial

import jax
import jax.numpy as jnp
from jax import lax
from jax.experimental import pallas as pl
from jax.experimental.pallas import tpu as pltpu
from jax.experimental.pallas import tpu_sc as plsc

N_NODES = 10000
N_GRAPHS = 128
NUM_EMB = 64
NUM_HEADS = 2

NC, NS, LANES = 2, 16, 16
NW = NC * NS
NPAD = 10240
ROWS_W = NPAD // NW
EDGE_SLACK = 1280


def _sc_mesh():
    return plsc.VectorSubcoreMesh(core_axis_name="c", subcore_axis_name="s")


def _sc_params():
    return pltpu.CompilerParams(use_tc_tiling_on_sc=False,
                                needs_layout_passes=False)


def _zero_acc(acc, rows, width):
    z = jnp.zeros((LANES,), jnp.float32)

    @pl.loop(0, rows)
    def _(r):
        for f in range(width // LANES):
            acc[r, pl.ds(f * LANES, LANES)] = z


def _pipeline(ch, lo, hi, idx_streams, gath_streams, isems, gsems, compute):
    lo8 = (lo // 8) * 8
    nj = (hi - lo8 + ch - 1) // ch
    npair = (nj + 1) // 2

    def fire_idx(p, base):
        for arr, buf in idx_streams:
            pltpu.make_async_copy(arr.at[pl.ds(base, ch)], buf.at[p], isems[p]).start()

    def wait_idx(p):
        for arr, buf in idx_streams:
            pltpu.make_async_copy(arr.at[pl.ds(0, ch)], buf.at[p], isems[p]).wait()

    def fire_gath(p):
        for tab, ibuf, dbuf in gath_streams:
            pltpu.make_async_copy(tab.at[ibuf.at[p]], dbuf.at[p], gsems[p]).start()

    def wait_gath(p):
        for tab, ibuf, dbuf in gath_streams:
            pltpu.make_async_copy(tab.at[ibuf.at[p]], dbuf.at[p], gsems[p]).wait()

    fire_idx(0, lo8)
    wait_idx(0)
    fire_gath(0)
    fire_idx(1, lo8 + ch)

    @pl.loop(0, npair)
    def _(jj):
        b0 = lo8 + (2 * jj) * ch
        for p in (0, 1):
            base = b0 + p * ch
            wait_idx(1 - p)
            fire_gath(1 - p)
            wait_gath(p)
            compute(p, base, lo, hi)
            fire_idx(p, base + 2 * ch)

    wait_gath(0)
    wait_idx(1)


def _flush_scatter(acc, out_hbm, oon2d, fidx, fsem, base32, n32):
    pltpu.sync_copy(oon2d.at[pl.ds(base32, n32)], fidx)
    cps = [pltpu.make_async_copy(acc.at[pl.ds(t * 32, 32)],
                                 out_hbm.at[fidx.at[t]], fsem)
           for t in range(n32)]
    for cp in cps:
        cp.start()
    for cp in cps:
        cp.wait()


GEN_CH = 256


def _gen_msg_body(o_hbm, es_hbm, src_hbm, dst_hbm, eid_hbm, off_hbm, oon_hbm,
                  out_hbm, sidx, didx, eidx, off_v, fidx, gbuf, ebuf, acc,
                  si0, si1, sg0, sg1, fsem):
    c = lax.axis_index("c")
    s = lax.axis_index("s")
    w = c * NS + s
    row0 = w * ROWS_W

    _zero_acc(acc, ROWS_W, 64)
    pltpu.sync_copy(off_hbm, off_v)
    iota = lax.iota(jnp.int32, LANES)
    ovec = plsc.load_gather(off_v, [w + iota])

    def compute(p, base, lo, hi):
        @pl.loop(0, GEN_CH // LANES)
        def _(g):
            dvec = jnp.clip(didx[p, pl.ds(g * LANES, LANES)] - row0, 0, ROWS_W - 1)
            for i in range(LANES):
                r = g * LANES + i
                pos = base + r
                valid = jnp.logical_and(pos >= lo, pos < hi)
                m = jnp.broadcast_to(valid, (LANES,))
                rowv = jnp.broadcast_to(dvec[i], (LANES,))
                for f in range(4):
                    sl = pl.ds(f * LANES, LANES)
                    vv = jnp.maximum(gbuf[p, r, sl] + ebuf[p, r, sl], 0.0) + 1e-7
                    plsc.addupdate_scatter(acc, [rowv, iota + f * LANES], vv, mask=m)

    _pipeline(GEN_CH, ovec[0], ovec[1],
              [(src_hbm, sidx), (dst_hbm, didx), (eid_hbm, eidx)],
              [(o_hbm, sidx, gbuf), (es_hbm, eidx, ebuf)],
              (si0, si1), (sg0, sg1), compute)

    _flush_scatter(acc, out_hbm, oon_hbm, fidx, fsem, w * 10, 10)


@jax.jit
def _sc_gen_msg(o_pad, e_store, src_s, dst_s, eid_s, offs, oon2d):
    k = pl.kernel(
        _gen_msg_body,
        out_type=jax.ShapeDtypeStruct((NPAD, 64), jnp.float32),
        mesh=_sc_mesh(),
        compiler_params=_sc_params(),
        scratch_types=[
            pltpu.VMEM((2, GEN_CH), jnp.int32),
            pltpu.VMEM((2, GEN_CH), jnp.int32),
            pltpu.VMEM((2, GEN_CH), jnp.int32),
            pltpu.VMEM((48,), jnp.int32),
            pltpu.VMEM((10, 32), jnp.int32),
            pltpu.VMEM((2, GEN_CH, 64), jnp.float32),
            pltpu.VMEM((2, GEN_CH, 64), jnp.float32),
            pltpu.VMEM((ROWS_W, 64), jnp.float32),
            pltpu.SemaphoreType.DMA,
            pltpu.SemaphoreType.DMA,
            pltpu.SemaphoreType.DMA,
            pltpu.SemaphoreType.DMA,
            pltpu.SemaphoreType.DMA,
        ],
    )
    return k(o_pad, e_store, src_s, dst_s, eid_s, offs, oon2d)


BC_CH = 64
HALF_W = ROWS_W // 2


def _attn_body(dtab, stab, es_hbm, src_hbm, dst_hbm, eid_hbm, off_hbm, oon_hbm,
               out_hbm, sidx, didx, eidx, off_v, fidx, dgath, sgath, ebuf, acc,
               si0, si1, sg0, sg1, fsem):
    c = lax.axis_index("c")
    s = lax.axis_index("s")
    w = c * NS + s

    pltpu.sync_copy(off_hbm, off_v)
    iota = lax.iota(jnp.int32, LANES)

    for half in range(2):
        row0 = w * ROWS_W + half * HALF_W
        _zero_acc(acc, HALF_W, 272)
        ovec = plsc.load_gather(off_v, [2 * w + half + iota])

        def compute(p, base, lo, hi):
            @pl.loop(0, BC_CH // LANES)
            def _(g):
                dvec = jnp.clip(didx[p, pl.ds(g * LANES, LANES)] - row0, 0, HALF_W - 1)
                for i in range(LANES):
                    r = g * LANES + i
                    pos = base + r
                    valid = jnp.logical_and(pos >= lo, pos < hi)
                    m = jnp.broadcast_to(valid, (LANES,))
                    rowv = jnp.broadcast_to(dvec[i], (LANES,))
                    qk = [dgath[p, r, pl.ds(f * LANES, LANES)] for f in range(8)]
                    qe = [dgath[p, r, pl.ds(128 + f * LANES, LANES)] for f in range(8)]
                    kv = [sgath[p, r, pl.ds(f * LANES, LANES)] for f in range(8)]
                    vv = [sgath[p, r, pl.ds(128 + f * LANES, LANES)] for f in range(8)]
                    ev = [ebuf[p, r, pl.ds(f * LANES, LANES)] for f in range(4)]
                    svec = dgath[p, r, pl.ds(256, LANES)]
                    m0 = qk[0] * kv[0]
                    m1 = qk[4] * kv[4]
                    for f in range(1, 4):
                        m0 = m0 + qk[f] * kv[f]
                        m1 = m1 + qk[4 + f] * kv[4 + f]
                    for f in range(4):
                        m0 = m0 + qe[f] * ev[f]
                        m1 = m1 + qe[4 + f] * ev[f]
                    a0 = jnp.sum(m0) * 0.125 - svec[0]
                    a1 = jnp.sum(m1) * 0.125 - svec[1]
                    zb0 = jnp.exp(jnp.broadcast_to(a0, (LANES,)))
                    zb1 = jnp.exp(jnp.broadcast_to(a1, (LANES,)))
                    for f in range(4):
                        sl = iota + f * LANES
                        plsc.addupdate_scatter(acc, [rowv, sl], vv[f] * zb0, mask=m)
                        plsc.addupdate_scatter(acc, [rowv, sl + 64], vv[4 + f] * zb1, mask=m)
                        plsc.addupdate_scatter(acc, [rowv, sl + 128], ev[f] * zb0, mask=m)
                        plsc.addupdate_scatter(acc, [rowv, sl + 192], ev[f] * zb1, mask=m)
                    zrow = jnp.where(iota == 0, zb0, jnp.where(iota == 1, zb1, 0.0))
                    plsc.addupdate_scatter(acc, [rowv, iota + 256], zrow, mask=m)

        _pipeline(BC_CH, ovec[0], ovec[1],
                  [(src_hbm, sidx), (dst_hbm, didx), (eid_hbm, eidx)],
                  [(dtab, didx, dgath), (stab, sidx, sgath), (es_hbm, eidx, ebuf)],
                  (si0, si1), (sg0, sg1), compute)

        _flush_scatter(acc, out_hbm, oon_hbm, fidx, fsem, w * 10 + half * 5, 5)


@jax.jit
def _sc_attn(dtab, stab, e_store, src_s, dst_s, eid_s, offs64, oon2d):
    k = pl.kernel(
        _attn_body,
        out_type=jax.ShapeDtypeStruct((NPAD, 272), jnp.float32),
        mesh=_sc_mesh(),
        compiler_params=_sc_params(),
        scratch_types=[
            pltpu.VMEM((2, BC_CH), jnp.int32),
            pltpu.VMEM((2, BC_CH), jnp.int32),
            pltpu.VMEM((2, BC_CH), jnp.int32),
            pltpu.VMEM((80,), jnp.int32),
            pltpu.VMEM((5, 32), jnp.int32),
            pltpu.VMEM((2, BC_CH, 272), jnp.float32),
            pltpu.VMEM((2, BC_CH, 256), jnp.float32),
            pltpu.VMEM((2, BC_CH, 64), jnp.float32),
            pltpu.VMEM((HALF_W, 272), jnp.float32),
            pltpu.SemaphoreType.DMA,
            pltpu.SemaphoreType.DMA,
            pltpu.SemaphoreType.DMA,
            pltpu.SemaphoreType.DMA,
            pltpu.SemaphoreType.DMA,
        ],
    )
    return k(dtab, stab, e_store, src_s, dst_s, eid_s, offs64, oon2d)


LA_CH = 256


def _lattr_body(es_hbm, dst_hbm, eid_hbm, off_hbm, oon_hbm, out_hbm,
                didx, eidx, off_v, fidx, ebuf, acc, si0, si1, sg0, sg1, fsem):
    c = lax.axis_index("c")
    s = lax.axis_index("s")
    w = c * NS + s
    row0 = w * ROWS_W

    _zero_acc(acc, ROWS_W, 80)
    pltpu.sync_copy(off_hbm, off_v)
    iota = lax.iota(jnp.int32, LANES)
    ovec = plsc.load_gather(off_v, [w + iota])
    onerow = jnp.where(iota == 0, 1.0, 0.0)

    def compute(p, base, lo, hi):
        @pl.loop(0, LA_CH // LANES)
        def _(g):
            dvec = jnp.clip(didx[p, pl.ds(g * LANES, LANES)] - row0, 0, ROWS_W - 1)
            for i in range(LANES):
                r = g * LANES + i
                pos = base + r
                valid = jnp.logical_and(pos >= lo, pos < hi)
                m = jnp.broadcast_to(valid, (LANES,))
                rowv = jnp.broadcast_to(dvec[i], (LANES,))
                for f in range(4):
                    plsc.addupdate_scatter(acc, [rowv, iota + f * LANES],
                                           ebuf[p, r, pl.ds(f * LANES, LANES)], mask=m)
                plsc.addupdate_scatter(acc, [rowv, iota + 64], onerow, mask=m)

    _pipeline(LA_CH, ovec[0], ovec[1],
              [(dst_hbm, didx), (eid_hbm, eidx)],
              [(es_hbm, eidx, ebuf)],
              (si0, si1), (sg0, sg1), compute)

    _flush_scatter(acc, out_hbm, oon_hbm, fidx, fsem, w * 10, 10)


@jax.jit
def _sc_loop_attr(e_store, dst_s, eid_s, offs, oon2d):
    k = pl.kernel(
        _lattr_body,
        out_type=jax.ShapeDtypeStruct((NPAD, 80), jnp.float32),
        mesh=_sc_mesh(),
        compiler_params=_sc_params(),
        scratch_types=[
            pltpu.VMEM((2, LA_CH), jnp.int32),
            pltpu.VMEM((2, LA_CH), jnp.int32),
            pltpu.VMEM((48,), jnp.int32),
            pltpu.VMEM((10, 32), jnp.int32),
            pltpu.VMEM((2, LA_CH, 64), jnp.float32),
            pltpu.VMEM((ROWS_W, 80), jnp.float32),
            pltpu.SemaphoreType.DMA,
            pltpu.SemaphoreType.DMA,
            pltpu.SemaphoreType.DMA,
            pltpu.SemaphoreType.DMA,
            pltpu.SemaphoreType.DMA,
        ],
    )
    return k(e_store, dst_s, eid_s, offs, oon2d)


def _leaky(x):
    return jnp.where(x >= 0, x, 0.01 * x)


def _mlp3_body(a_ref, w1, b1, w2, b2, w3, b3, o_ref):
    a = a_ref[...]
    h = _leaky(jnp.dot(a, w1[...], preferred_element_type=jnp.float32) + b1[...])
    h = _leaky(jnp.dot(h, w2[...], preferred_element_type=jnp.float32) + b2[...])
    h = jnp.dot(h, w3[...], preferred_element_type=jnp.float32) + b3[...]
    o_ref[...] = h


def _mlp3(a, p, bm=2048):
    M, F = a.shape
    D = p["Ws"][2].shape[1]
    grid = (pl.cdiv(M, bm),)
    full = lambda shape: pl.BlockSpec(shape, lambda i: (0,) * len(shape))
    return pl.pallas_call(
        _mlp3_body,
        grid=grid,
        in_specs=[
            pl.BlockSpec((bm, F), lambda i: (i, 0)),
            full(p["Ws"][0].shape), full((1, p["bs"][0].shape[0])),
            full(p["Ws"][1].shape), full((1, p["bs"][1].shape[0])),
            full(p["Ws"][2].shape), full((1, p["bs"][2].shape[0])),
        ],
        out_specs=pl.BlockSpec((bm, D), lambda i: (i, 0)),
        out_shape=jax.ShapeDtypeStruct((M, D), jnp.float32),
    )(a, p["Ws"][0], p["bs"][0][None], p["Ws"][1], p["bs"][1][None],
      p["Ws"][2], p["bs"][2][None])


BM = 1024


def _full(shape):
    return pl.BlockSpec(shape, lambda i: (0,) * len(shape))


def _rows(width):
    return pl.BlockSpec((BM, width), lambda i: (i, 0))


def _t1_body(o_ref, agg_ref, la_ref, genW, genb, wqkvs, bqkvs, weT2, we,
             dtab_ref, stab_ref, skip_ref):
    o = o_ref[...]
    genout = jnp.dot(agg_ref[...] + o, genW[...],
                     preferred_element_type=jnp.float32) + genb[...]
    xc = jnp.concatenate([o, genout], axis=1)
    qkvs = jnp.dot(xc, wqkvs[...], preferred_element_type=jnp.float32) + bqkvs[...]
    q = qkvs[:, 0:128]
    k = qkvs[:, 128:256]
    vv = qkvs[:, 256:384]
    skip_ref[...] = qkvs[:, 384:512]
    qWe = jnp.dot(q, weT2[...], preferred_element_type=jnp.float32)
    elC = jnp.dot(la_ref[...], we[...], preferred_element_type=jnp.float32)
    kc = k + elC
    s0 = (q[:, :64] * kc[:, :64]).sum(axis=1, keepdims=True) * 0.125
    s1 = (q[:, 64:] * kc[:, 64:]).sum(axis=1, keepdims=True) * 0.125
    dtab_ref[...] = jnp.concatenate(
        [q, qWe, s0, s1, jnp.zeros((o.shape[0], 14), jnp.float32)], axis=1)
    stab_ref[...] = jnp.concatenate([k, vv], axis=1)


def _t1(o, agg, la_pad, p):
    wqkvs = jnp.concatenate([p["Wq"], p["Wk"], p["Wv"], p["Wsk"]], axis=1)
    bqkvs = jnp.concatenate([p["bq"], p["bk"], p["bv"], p["bsk"]])[None]
    z64 = jnp.zeros((64, 64), jnp.float32)
    weT2 = jnp.concatenate([
        jnp.concatenate([p["We"][:, :64].T, z64], axis=1),
        jnp.concatenate([z64, p["We"][:, 64:].T], axis=1)], axis=0)
    return pl.pallas_call(
        _t1_body,
        grid=(NPAD // BM,),
        in_specs=[_rows(64), _rows(64), _rows(64),
                  _full((64, 64)), _full((1, 64)), _full((128, 512)),
                  _full((1, 512)), _full((128, 128)), _full((64, 128))],
        out_specs=[_rows(272), _rows(256), _rows(128)],
        out_shape=[jax.ShapeDtypeStruct((NPAD, 272), jnp.float32),
                   jax.ShapeDtypeStruct((NPAD, 256), jnp.float32),
                   jax.ShapeDtypeStruct((NPAD, 128), jnp.float32)],
    )(o, agg, la_pad, p["gen_W"], p["gen_b"][None], wqkvs, bqkvs, weT2, p["We"])


def _t2_body(acc_ref, skip_ref, o_ref, we0, we1, linW, linb, pt_ref,
             y_ref, stats_ref):
    i = pl.program_id(0)
    a = acc_ref[...]
    out0 = (a[:, 0:64] + jnp.dot(a[:, 128:192], we0[...],
                                 preferred_element_type=jnp.float32)) / a[:, 256:257]
    out1 = (a[:, 64:128] + jnp.dot(a[:, 192:256], we1[...],
                                   preferred_element_type=jnp.float32)) / a[:, 257:258]
    t = jnp.concatenate([out0, out1], axis=1) + skip_ref[...]
    y = o_ref[...] + jnp.dot(t, linW[...], preferred_element_type=jnp.float32) + linb[...]
    y_ref[...] = y
    yy = jnp.concatenate([y, y * y, jnp.ones((y.shape[0], 16), jnp.float32)], axis=1)
    st = jnp.dot(pt_ref[...], yy, preferred_element_type=jnp.float32)

    @pl.when(i == 0)
    def _():
        stats_ref[...] = jnp.zeros_like(stats_ref)

    stats_ref[...] += st


def _t2(accA, skip, o, pt, p):
    return pl.pallas_call(
        _t2_body,
        grid=(NPAD // BM,),
        in_specs=[_rows(272), _rows(128), _rows(64),
                  _full((64, 64)), _full((64, 64)), _full((128, 64)),
                  _full((1, 64)), pl.BlockSpec((128, BM), lambda i: (0, i))],
        out_specs=[_rows(64), _full((128, 144))],
        out_shape=[jax.ShapeDtypeStruct((NPAD, 64), jnp.float32),
                   jax.ShapeDtypeStruct((128, 144), jnp.float32)],
    )(accA, skip, o, p["We"][:, :64], p["We"][:, 64:], p["lin_W"],
      p["lin_b"][None], pt)


def _stats_to_mi(stats, eps=1e-5):
    s1 = stats[:, 0:64].sum(axis=1)
    s2 = stats[:, 64:128].sum(axis=1)
    cnt = stats[:, 128]
    norm = jnp.maximum(cnt, 1.0) * 64.0
    mean = s1 / norm
    var = s2 / norm - mean * mean
    inv = 1.0 / jnp.sqrt(var + eps)
    z = jnp.zeros((128, 126), jnp.float32)
    return jnp.concatenate([mean[:, None], inv[:, None], z], axis=1)


def _t3_body(y_ref, stats_ref, pb_ref, pt_ref, w1, b1, w2, b2,
             y2_ref, stats2_ref):
    i = pl.program_id(0)
    mi = jnp.dot(pb_ref[...], _stats_to_mi(stats_ref[...]),
                 preferred_element_type=jnp.float32)
    o1 = (y_ref[...] - mi[:, 0:1]) * mi[:, 1:2]
    h = _leaky(jnp.dot(o1, w1[...], preferred_element_type=jnp.float32) + b1[...])
    y2 = o1 + jnp.dot(h, w2[...], preferred_element_type=jnp.float32) + b2[...]
    y2_ref[...] = y2
    yy = jnp.concatenate([y2, y2 * y2, jnp.ones((y2.shape[0], 16), jnp.float32)], axis=1)
    st = jnp.dot(pt_ref[...], yy, preferred_element_type=jnp.float32)

    @pl.when(i == 0)
    def _():
        stats2_ref[...] = jnp.zeros_like(stats2_ref)

    stats2_ref[...] += st


def _t3(y, stats, pmat, pt, p):
    return pl.pallas_call(
        _t3_body,
        grid=(NPAD // BM,),
        in_specs=[_rows(64), _full((128, 144)), _rows(128),
                  pl.BlockSpec((128, BM), lambda i: (0, i)),
                  _full((64, 256)), _full((1, 256)), _full((256, 64)),
                  _full((1, 64))],
        out_specs=[_rows(64), _full((128, 144))],
        out_shape=[jax.ShapeDtypeStruct((NPAD, 64), jnp.float32),
                   jax.ShapeDtypeStruct((128, 144), jnp.float32)],
    )(y, stats, pmat, pt, p["ff"]["Ws"][0], p["ff"]["bs"][0][None],
      p["ff"]["Ws"][1], p["ff"]["bs"][1][None])


def _t4_body(y2_ref, stats_ref, pb_ref, o_ref):
    mi = jnp.dot(pb_ref[...], _stats_to_mi(stats_ref[...]),
                 preferred_element_type=jnp.float32)
    o_ref[...] = (y2_ref[...] - mi[:, 0:1]) * mi[:, 1:2]


def _t4(y2, stats2, pmat):
    return pl.pallas_call(
        _t4_body,
        grid=(NPAD // BM,),
        in_specs=[_rows(64), _full((128, 144)), _rows(128)],
        out_specs=_rows(64),
        out_shape=jax.ShapeDtypeStruct((NPAD, 64), jnp.float32),
    )(y2, stats2, pmat)


def _tfa_body(o_ref, cr_ref, pt_ref, m_ref):
    i = pl.program_id(0)
    yy = jnp.concatenate([o_ref[...], cr_ref[...]], axis=1)
    st = jnp.dot(pt_ref[...], yy, preferred_element_type=jnp.float32)

    @pl.when(i == 0)
    def _():
        m_ref[...] = jnp.zeros_like(m_ref)

    m_ref[...] += st


def _tfa(o, colreal, pt):
    return pl.pallas_call(
        _tfa_body,
        grid=(NPAD // BM,),
        in_specs=[_rows(64), _rows(16), pl.BlockSpec((128, BM), lambda i: (0, i))],
        out_specs=_full((128, 80)),
        out_shape=jax.ShapeDtypeStruct((128, 80), jnp.float32),
    )(o, colreal, pt)


def _tfb_body(m_ref, ov_ref, c_ref, glob_ref):
    m = m_ref[...]
    ov = ov_ref[...]
    gmp = (m[:, 0:64] - ov) / jnp.maximum(m[:, 64:65], 1.0)
    glob_ref[...] = jnp.concatenate([gmp, ov, c_ref[...]], axis=1)


def _tfb(m, o_virt, c):
    return pl.pallas_call(
        _tfb_body,
        grid=(1,),
        in_specs=[_full((128, 80)), _full((128, 64)), _full((128, 64))],
        out_specs=_full((128, 192)),
        out_shape=jax.ShapeDtypeStruct((128, 192), jnp.float32),
    )(m, o_virt, c)


def _tfc_body(o_ref, pb_ref, c_ref, out_ref):
    cb = jnp.dot(pb_ref[...], c_ref[...], preferred_element_type=jnp.float32)
    out_ref[...] = jnp.concatenate([o_ref[...], cb], axis=1)


def _tfc(o, pmat, c, n):
    bm = 1000
    return pl.pallas_call(
        _tfc_body,
        grid=(n // bm,),
        in_specs=[pl.BlockSpec((bm, 64), lambda i: (i, 0)),
                  pl.BlockSpec((bm, 128), lambda i: (i, 0)),
                  _full((128, 64))],
        out_specs=pl.BlockSpec((bm, 128), lambda i: (i, 0)),
        out_shape=jax.ShapeDtypeStruct((n, 128), jnp.float32),
    )(o, pmat, c)


def _ldiv_body(la_ref, out_ref):
    la = la_ref[...]
    out_ref[...] = la[:, 0:64] / jnp.maximum(la[:, 64:65], 1.0)


def _ldiv(la):
    return pl.pallas_call(
        _ldiv_body,
        grid=(NPAD // BM,),
        in_specs=[_rows(80)],
        out_specs=_rows(64),
        out_shape=jax.ShapeDtypeStruct((NPAD, 64), jnp.float32),
    )(la)


def kernel(x, edge_index, edge_attr, batch, cond, params):
    N, G = x.shape[0], cond.shape[0]
    o = _mlp3(x, params["x2h"])
    e = _mlp3(edge_attr, params["e2h"])
    c = _mlp3(cond, params["c2h"], bm=128)

    u = jnp.arange(N, dtype=edge_index.dtype)
    v = batch.astype(edge_index.dtype) + N
    n_total = N + G
    NE = edge_index.shape[1]
    o = jnp.concatenate([o, c,
                         jnp.zeros((NPAD - n_total, 64), jnp.float32)], axis=0)

    rn = jnp.arange(NPAD)
    batch_pad = jnp.concatenate([
        batch.astype(jnp.int32), jnp.arange(G, dtype=jnp.int32),
        jnp.zeros((NPAD - n_total,), jnp.int32)])
    pmat = ((batch_pad[:, None] == jnp.arange(G, dtype=jnp.int32)[None, :])
            & (rn[:, None] < n_total)).astype(jnp.float32)
    pt = pmat.T
    colreal = ((rn[:, None] < N)
               & (jnp.arange(16)[None, :] == 0)).astype(jnp.float32)

    ar_n = jnp.arange(N, dtype=jnp.int32)
    new_of_old = jnp.concatenate([ar_n + 1 + ar_n // 79,
                                  80 * jnp.arange(G, dtype=jnp.int32)])
    arr = jnp.full((NPAD,), -1, jnp.int32).at[new_of_old].set(
        jnp.arange(n_total, dtype=jnp.int32))
    unused = arr < 0
    old_of_new = jnp.where(unused,
                           n_total + jnp.cumsum(unused.astype(jnp.int32)) - 1,
                           arr)
    oon2d = old_of_new.reshape(NPAD // 32, 32)

    def _sortset(srcs, dsts, eids, grans):
        E = dsts.shape[0]
        E_pad = 128 * pl.cdiv(E, 128) + EDGE_SLACK
        pad_e = E_pad - E
        dstsN = new_of_old[dsts]
        perm = jnp.argsort(dstsN)

        def _padi(a):
            return jnp.concatenate([a.astype(jnp.int32),
                                    jnp.zeros((pad_e,), jnp.int32)])

        dst_sorted = dstsN[perm]
        offlist = []
        for gran in grans:
            nb = NPAD // gran
            offs = jnp.searchsorted(dst_sorted,
                                    jnp.arange(nb + 1, dtype=jnp.int32) * gran)
            npadofs = 16 * pl.cdiv(nb + 16, 16)
            offlist.append(jnp.concatenate([
                offs.astype(jnp.int32),
                jnp.full((npadofs - nb - 1,), E, jnp.int32)]))
        src_p = _padi(srcs[perm]) if srcs is not None else None
        return src_p, _padi(dst_sorted), _padi(eids[perm]), offlist

    src0 = jnp.concatenate([edge_index[0], u, v])
    dst0 = jnp.concatenate([edge_index[1], v, u])
    eid0 = jnp.concatenate([jnp.arange(NE, dtype=jnp.int32),
                            jnp.full((2 * N,), NE, jnp.int32)])
    _, dst0_s, eid0_s, (offs0,) = _sortset(None, dst0, eid0, [ROWS_W])

    e_p_row = jnp.zeros((1, 64), jnp.float32).at[0, 0].set(1.0)
    e_store0 = jnp.concatenate([e, e_p_row,
                                jnp.zeros((n_total + 7, 64), jnp.float32)])
    la = _sc_loop_attr(e_store0, dst0_s, eid0_s, offs0, oon2d)
    loop_attr = _ldiv(la)
    e_store = lax.dynamic_update_slice(e_store0, loop_attr[:n_total], (NE + 1, 0))

    sl = jnp.arange(n_total, dtype=edge_index.dtype)
    aug_src = jnp.concatenate([src0, sl])
    aug_dst = jnp.concatenate([dst0, sl])
    eid = jnp.concatenate([eid0, NE + 1 + jnp.arange(n_total, dtype=jnp.int32)])
    src_s, dst_s, eid_s, (offs, offs64) = _sortset(aug_src, aug_dst, eid,
                                                   [ROWS_W, HALF_W])

    for p in params["layers"]:
        agg = _sc_gen_msg(o, e_store, src_s, dst_s, eid_s, offs, oon2d)
        dtab, stab, skip = _t1(o, agg, loop_attr, p)
        accA = _sc_attn(dtab, stab, e_store, src_s, dst_s, eid_s, offs64, oon2d)
        y, stats = _t2(accA, skip, o, pt, p)
        y2, stats2 = _t3(y, stats, pmat, pt, p)
        o = _t4(y2, stats2, pmat)

    m = _tfa(o, colreal, pt)
    o_virt = o[N:N + G]
    glob = _tfb(m, o_virt, c)
    o_final = _tfc(o, pmat, c, N)
    return (o_final, glob)


# speedup vs baseline: 1.3193x; 1.3193x over previous
import math
from functools import partial

import jax
import jax.numpy as jnp
from jax import lax
from jax.experimental import pallas as pl
from jax.experimental.pallas import tpu as pltpu
from jax.experimental.pallas import tpu_sc as plsc

N_NODES = 10000
N_GRAPHS = 128
NUM_EMB = 64
NUM_HEADS = 2

NC, NS, LANES = 2, 16, 16
NW = NC * NS
NPAD = 10240
ROWS_W = NPAD // NW
EDGE_SLACK = 1280


def _sc_mesh():
    return plsc.VectorSubcoreMesh(core_axis_name="c", subcore_axis_name="s")


def _sc_params():
    return pltpu.CompilerParams(use_tc_tiling_on_sc=False,
                                needs_layout_passes=False)


def _zero_acc(acc, rows, width):
    z = jnp.zeros((LANES,), jnp.float32)

    @pl.loop(0, rows)
    def _(r):
        for f in range(width // LANES):
            acc[r, pl.ds(f * LANES, LANES)] = z


def _pipeline(ch, lo, hi, idx_streams, gath_streams, isems, gsems, compute):
    lo8 = (lo // 8) * 8
    nj = (hi - lo8 + ch - 1) // ch
    npair = (nj + 1) // 2

    def fire_idx(p, base):
        for arr, buf in idx_streams:
            pltpu.make_async_copy(arr.at[pl.ds(base, ch)], buf.at[p], isems[p]).start()

    def wait_idx(p):
        for arr, buf in idx_streams:
            pltpu.make_async_copy(arr.at[pl.ds(0, ch)], buf.at[p], isems[p]).wait()

    def fire_gath(p):
        for tab, ibuf, dbuf in gath_streams:
            pltpu.make_async_copy(tab.at[ibuf.at[p]], dbuf.at[p], gsems[p]).start()

    def wait_gath(p):
        for tab, ibuf, dbuf in gath_streams:
            pltpu.make_async_copy(tab.at[ibuf.at[p]], dbuf.at[p], gsems[p]).wait()

    fire_idx(0, lo8)
    wait_idx(0)
    fire_gath(0)
    fire_idx(1, lo8 + ch)

    @pl.loop(0, npair)
    def _(jj):
        b0 = lo8 + (2 * jj) * ch
        for p in (0, 1):
            base = b0 + p * ch
            wait_idx(1 - p)
            fire_gath(1 - p)
            wait_gath(p)
            compute(p, base, lo, hi)
            fire_idx(p, base + 2 * ch)

    wait_gath(0)
    wait_idx(1)


def _flush_scatter(acc, out_hbm, oon2d, fidx, fsem, base32, n32):
    pltpu.sync_copy(oon2d.at[pl.ds(base32, n32)], fidx)
    cps = [pltpu.make_async_copy(acc.at[pl.ds(t * 32, 32)],
                                 out_hbm.at[fidx.at[t]], fsem)
           for t in range(n32)]
    for cp in cps:
        cp.start()
    for cp in cps:
        cp.wait()


GEN_CH = 256


def _gen_msg_body(o_hbm, es_hbm, src_hbm, dst_hbm, eid_hbm, off_hbm, oon_hbm,
                  out_hbm, sidx, didx, eidx, off_v, fidx, gbuf, ebuf, acc,
                  si0, si1, sg0, sg1, fsem):
    c = lax.axis_index("c")
    s = lax.axis_index("s")
    w = c * NS + s
    row0 = w * ROWS_W

    _zero_acc(acc, ROWS_W, 64)
    pltpu.sync_copy(off_hbm, off_v)
    iota = lax.iota(jnp.int32, LANES)
    ovec = plsc.load_gather(off_v, [w + iota])

    def compute(p, base, lo, hi):
        @pl.loop(0, GEN_CH // LANES)
        def _(g):
            dvec = jnp.clip(didx[p, pl.ds(g * LANES, LANES)] - row0, 0, ROWS_W - 1)
            for i in range(LANES):
                r = g * LANES + i
                pos = base + r
                valid = jnp.logical_and(pos >= lo, pos < hi)
                m = jnp.broadcast_to(valid, (LANES,))
                rowv = jnp.broadcast_to(dvec[i], (LANES,))
                for f in range(4):
                    sl = pl.ds(f * LANES, LANES)
                    vv = jnp.maximum(gbuf[p, r, sl] + ebuf[p, r, sl], 0.0) + 1e-7
                    plsc.addupdate_scatter(acc, [rowv, iota + f * LANES], vv, mask=m)

    _pipeline(GEN_CH, ovec[0], ovec[1],
              [(src_hbm, sidx), (dst_hbm, didx), (eid_hbm, eidx)],
              [(o_hbm, sidx, gbuf), (es_hbm, eidx, ebuf)],
              (si0, si1), (sg0, sg1), compute)

    _flush_scatter(acc, out_hbm, oon_hbm, fidx, fsem, w * 10, 10)


@jax.jit
def _sc_gen_msg(o_pad, e_store, src_s, dst_s, eid_s, offs, oon2d):
    k = pl.kernel(
        _gen_msg_body,
        out_type=jax.ShapeDtypeStruct((NPAD, 64), jnp.float32),
        mesh=_sc_mesh(),
        compiler_params=_sc_params(),
        scratch_types=[
            pltpu.VMEM((2, GEN_CH), jnp.int32),
            pltpu.VMEM((2, GEN_CH), jnp.int32),
            pltpu.VMEM((2, GEN_CH), jnp.int32),
            pltpu.VMEM((48,), jnp.int32),
            pltpu.VMEM((10, 32), jnp.int32),
            pltpu.VMEM((2, GEN_CH, 64), jnp.float32),
            pltpu.VMEM((2, GEN_CH, 64), jnp.float32),
            pltpu.VMEM((ROWS_W, 64), jnp.float32),
            pltpu.SemaphoreType.DMA,
            pltpu.SemaphoreType.DMA,
            pltpu.SemaphoreType.DMA,
            pltpu.SemaphoreType.DMA,
            pltpu.SemaphoreType.DMA,
        ],
    )
    return k(o_pad, e_store, src_s, dst_s, eid_s, offs, oon2d)


BC_CH = 64
HALF_W = ROWS_W // 2


def _attn_body(dtab, stab, es_hbm, src_hbm, dst_hbm, eid_hbm, off_hbm, oon_hbm,
               out_hbm, sidx, didx, eidx, off_v, fidx, dgath, sgath, ebuf, acc,
               si0, si1, sg0, sg1, fsem):
    c = lax.axis_index("c")
    s = lax.axis_index("s")
    w = c * NS + s

    pltpu.sync_copy(off_hbm, off_v)
    iota = lax.iota(jnp.int32, LANES)

    for half in range(2):
        row0 = w * ROWS_W + half * HALF_W
        _zero_acc(acc, HALF_W, 272)
        ovec = plsc.load_gather(off_v, [2 * w + half + iota])

        def compute(p, base, lo, hi):
            @pl.loop(0, BC_CH // LANES)
            def _(g):
                dvec = jnp.clip(didx[p, pl.ds(g * LANES, LANES)] - row0, 0, HALF_W - 1)
                for i in range(LANES):
                    r = g * LANES + i
                    pos = base + r
                    valid = jnp.logical_and(pos >= lo, pos < hi)
                    m = jnp.broadcast_to(valid, (LANES,))
                    rowv = jnp.broadcast_to(dvec[i], (LANES,))
                    qk = [dgath[p, r, pl.ds(f * LANES, LANES)] for f in range(8)]
                    qe = [dgath[p, r, pl.ds(128 + f * LANES, LANES)] for f in range(8)]
                    kv = [sgath[p, r, pl.ds(f * LANES, LANES)] for f in range(8)]
                    vv = [sgath[p, r, pl.ds(128 + f * LANES, LANES)] for f in range(8)]
                    ev = [ebuf[p, r, pl.ds(f * LANES, LANES)] for f in range(4)]
                    svec = dgath[p, r, pl.ds(256, LANES)]
                    m0 = qk[0] * kv[0]
                    m1 = qk[4] * kv[4]
                    for f in range(1, 4):
                        m0 = m0 + qk[f] * kv[f]
                        m1 = m1 + qk[4 + f] * kv[4 + f]
                    for f in range(4):
                        m0 = m0 + qe[f] * ev[f]
                        m1 = m1 + qe[4 + f] * ev[f]
                    a0 = jnp.sum(m0) * 0.125 - svec[0]
                    a1 = jnp.sum(m1) * 0.125 - svec[1]
                    zb0 = jnp.exp(jnp.broadcast_to(a0, (LANES,)))
                    zb1 = jnp.exp(jnp.broadcast_to(a1, (LANES,)))
                    for f in range(4):
                        sl = iota + f * LANES
                        plsc.addupdate_scatter(acc, [rowv, sl], vv[f] * zb0, mask=m)
                        plsc.addupdate_scatter(acc, [rowv, sl + 64], vv[4 + f] * zb1, mask=m)
                        plsc.addupdate_scatter(acc, [rowv, sl + 128], ev[f] * zb0, mask=m)
                        plsc.addupdate_scatter(acc, [rowv, sl + 192], ev[f] * zb1, mask=m)
                    zrow = jnp.where(iota == 0, zb0, jnp.where(iota == 1, zb1, 0.0))
                    plsc.addupdate_scatter(acc, [rowv, iota + 256], zrow, mask=m)

        _pipeline(BC_CH, ovec[0], ovec[1],
                  [(src_hbm, sidx), (dst_hbm, didx), (eid_hbm, eidx)],
                  [(dtab, didx, dgath), (stab, sidx, sgath), (es_hbm, eidx, ebuf)],
                  (si0, si1), (sg0, sg1), compute)

        _flush_scatter(acc, out_hbm, oon_hbm, fidx, fsem, w * 10 + half * 5, 5)


@jax.jit
def _sc_attn(dtab, stab, e_store, src_s, dst_s, eid_s, offs64, oon2d):
    k = pl.kernel(
        _attn_body,
        out_type=jax.ShapeDtypeStruct((NPAD, 272), jnp.float32),
        mesh=_sc_mesh(),
        compiler_params=_sc_params(),
        scratch_types=[
            pltpu.VMEM((2, BC_CH), jnp.int32),
            pltpu.VMEM((2, BC_CH), jnp.int32),
            pltpu.VMEM((2, BC_CH), jnp.int32),
            pltpu.VMEM((80,), jnp.int32),
            pltpu.VMEM((5, 32), jnp.int32),
            pltpu.VMEM((2, BC_CH, 272), jnp.float32),
            pltpu.VMEM((2, BC_CH, 256), jnp.float32),
            pltpu.VMEM((2, BC_CH, 64), jnp.float32),
            pltpu.VMEM((HALF_W, 272), jnp.float32),
            pltpu.SemaphoreType.DMA,
            pltpu.SemaphoreType.DMA,
            pltpu.SemaphoreType.DMA,
            pltpu.SemaphoreType.DMA,
            pltpu.SemaphoreType.DMA,
        ],
    )
    return k(dtab, stab, e_store, src_s, dst_s, eid_s, offs64, oon2d)


LA_CH = 256


def _lattr_body(es_hbm, dst_hbm, eid_hbm, off_hbm, oon_hbm, out_hbm,
                didx, eidx, off_v, fidx, ebuf, acc, si0, si1, sg0, sg1, fsem):
    c = lax.axis_index("c")
    s = lax.axis_index("s")
    w = c * NS + s
    row0 = w * ROWS_W

    _zero_acc(acc, ROWS_W, 80)
    pltpu.sync_copy(off_hbm, off_v)
    iota = lax.iota(jnp.int32, LANES)
    ovec = plsc.load_gather(off_v, [w + iota])
    onerow = jnp.where(iota == 0, 1.0, 0.0)

    def compute(p, base, lo, hi):
        @pl.loop(0, LA_CH // LANES)
        def _(g):
            dvec = jnp.clip(didx[p, pl.ds(g * LANES, LANES)] - row0, 0, ROWS_W - 1)
            for i in range(LANES):
                r = g * LANES + i
                pos = base + r
                valid = jnp.logical_and(pos >= lo, pos < hi)
                m = jnp.broadcast_to(valid, (LANES,))
                rowv = jnp.broadcast_to(dvec[i], (LANES,))
                for f in range(4):
                    plsc.addupdate_scatter(acc, [rowv, iota + f * LANES],
                                           ebuf[p, r, pl.ds(f * LANES, LANES)], mask=m)
                plsc.addupdate_scatter(acc, [rowv, iota + 64], onerow, mask=m)

    _pipeline(LA_CH, ovec[0], ovec[1],
              [(dst_hbm, didx), (eid_hbm, eidx)],
              [(es_hbm, eidx, ebuf)],
              (si0, si1), (sg0, sg1), compute)

    _flush_scatter(acc, out_hbm, oon_hbm, fidx, fsem, w * 10, 10)


@jax.jit
def _sc_loop_attr(e_store, dst_s, eid_s, offs, oon2d):
    k = pl.kernel(
        _lattr_body,
        out_type=jax.ShapeDtypeStruct((NPAD, 80), jnp.float32),
        mesh=_sc_mesh(),
        compiler_params=_sc_params(),
        scratch_types=[
            pltpu.VMEM((2, LA_CH), jnp.int32),
            pltpu.VMEM((2, LA_CH), jnp.int32),
            pltpu.VMEM((48,), jnp.int32),
            pltpu.VMEM((10, 32), jnp.int32),
            pltpu.VMEM((2, LA_CH, 64), jnp.float32),
            pltpu.VMEM((ROWS_W, 80), jnp.float32),
            pltpu.SemaphoreType.DMA,
            pltpu.SemaphoreType.DMA,
            pltpu.SemaphoreType.DMA,
            pltpu.SemaphoreType.DMA,
            pltpu.SemaphoreType.DMA,
        ],
    )
    return k(e_store, dst_s, eid_s, offs, oon2d)


def _leaky(x):
    return jnp.where(x >= 0, x, 0.01 * x)


def _mlp3_body(a_ref, w1, b1, w2, b2, w3, b3, o_ref):
    a = a_ref[...]
    h = _leaky(jnp.dot(a, w1[...], preferred_element_type=jnp.float32) + b1[...])
    h = _leaky(jnp.dot(h, w2[...], preferred_element_type=jnp.float32) + b2[...])
    h = jnp.dot(h, w3[...], preferred_element_type=jnp.float32) + b3[...]
    o_ref[...] = h


def _mlp3(a, p, bm=2048):
    M, F = a.shape
    D = p["Ws"][2].shape[1]
    grid = (pl.cdiv(M, bm),)
    full = lambda shape: pl.BlockSpec(shape, lambda i: (0,) * len(shape))
    return pl.pallas_call(
        _mlp3_body,
        grid=grid,
        in_specs=[
            pl.BlockSpec((bm, F), lambda i: (i, 0)),
            full(p["Ws"][0].shape), full((1, p["bs"][0].shape[0])),
            full(p["Ws"][1].shape), full((1, p["bs"][1].shape[0])),
            full(p["Ws"][2].shape), full((1, p["bs"][2].shape[0])),
        ],
        out_specs=pl.BlockSpec((bm, D), lambda i: (i, 0)),
        out_shape=jax.ShapeDtypeStruct((M, D), jnp.float32),
    )(a, p["Ws"][0], p["bs"][0][None], p["Ws"][1], p["bs"][1][None],
      p["Ws"][2], p["bs"][2][None])


BM = 1024


def _full(shape):
    return pl.BlockSpec(shape, lambda i: (0,) * len(shape))


def _rows(width):
    return pl.BlockSpec((BM, width), lambda i: (i, 0))


def _t1_body(o_ref, agg_ref, la_ref, genW, genb, wqkvs, bqkvs, weT2, we,
             dtab_ref, stab_ref, skip_ref):
    o = o_ref[...]
    genout = jnp.dot(agg_ref[...] + o, genW[...],
                     preferred_element_type=jnp.float32) + genb[...]
    xc = jnp.concatenate([o, genout], axis=1)
    qkvs = jnp.dot(xc, wqkvs[...], preferred_element_type=jnp.float32) + bqkvs[...]
    q = qkvs[:, 0:128]
    k = qkvs[:, 128:256]
    vv = qkvs[:, 256:384]
    skip_ref[...] = qkvs[:, 384:512]
    qWe = jnp.dot(q, weT2[...], preferred_element_type=jnp.float32)
    elC = jnp.dot(la_ref[...], we[...], preferred_element_type=jnp.float32)
    kc = k + elC
    s0 = (q[:, :64] * kc[:, :64]).sum(axis=1, keepdims=True) * 0.125
    s1 = (q[:, 64:] * kc[:, 64:]).sum(axis=1, keepdims=True) * 0.125
    dtab_ref[...] = jnp.concatenate(
        [q, qWe, s0, s1, jnp.zeros((o.shape[0], 14), jnp.float32)], axis=1)
    stab_ref[...] = jnp.concatenate([k, vv], axis=1)


def _t1(o, agg, la_pad, p):
    wqkvs = jnp.concatenate([p["Wq"], p["Wk"], p["Wv"], p["Wsk"]], axis=1)
    bqkvs = jnp.concatenate([p["bq"], p["bk"], p["bv"], p["bsk"]])[None]
    z64 = jnp.zeros((64, 64), jnp.float32)
    weT2 = jnp.concatenate([
        jnp.concatenate([p["We"][:, :64].T, z64], axis=1),
        jnp.concatenate([z64, p["We"][:, 64:].T], axis=1)], axis=0)
    return pl.pallas_call(
        _t1_body,
        grid=(NPAD // BM,),
        in_specs=[_rows(64), _rows(64), _rows(64),
                  _full((64, 64)), _full((1, 64)), _full((128, 512)),
                  _full((1, 512)), _full((128, 128)), _full((64, 128))],
        out_specs=[_rows(272), _rows(256), _rows(128)],
        out_shape=[jax.ShapeDtypeStruct((NPAD, 272), jnp.float32),
                   jax.ShapeDtypeStruct((NPAD, 256), jnp.float32),
                   jax.ShapeDtypeStruct((NPAD, 128), jnp.float32)],
    )(o, agg, la_pad, p["gen_W"], p["gen_b"][None], wqkvs, bqkvs, weT2, p["We"])


def _t2_body(acc_ref, skip_ref, o_ref, we0, we1, linW, linb, pt_ref,
             y_ref, stats_ref):
    i = pl.program_id(0)
    a = acc_ref[...]
    out0 = (a[:, 0:64] + jnp.dot(a[:, 128:192], we0[...],
                                 preferred_element_type=jnp.float32)) / a[:, 256:257]
    out1 = (a[:, 64:128] + jnp.dot(a[:, 192:256], we1[...],
                                   preferred_element_type=jnp.float32)) / a[:, 257:258]
    t = jnp.concatenate([out0, out1], axis=1) + skip_ref[...]
    y = o_ref[...] + jnp.dot(t, linW[...], preferred_element_type=jnp.float32) + linb[...]
    y_ref[...] = y
    yy = jnp.concatenate([y, y * y, jnp.ones((y.shape[0], 16), jnp.float32)], axis=1)
    st = jnp.dot(pt_ref[...], yy, preferred_element_type=jnp.float32)

    @pl.when(i == 0)
    def _():
        stats_ref[...] = jnp.zeros_like(stats_ref)

    stats_ref[...] += st


def _t2(accA, skip, o, pt, p):
    return pl.pallas_call(
        _t2_body,
        grid=(NPAD // BM,),
        in_specs=[_rows(272), _rows(128), _rows(64),
                  _full((64, 64)), _full((64, 64)), _full((128, 64)),
                  _full((1, 64)), pl.BlockSpec((128, BM), lambda i: (0, i))],
        out_specs=[_rows(64), _full((128, 144))],
        out_shape=[jax.ShapeDtypeStruct((NPAD, 64), jnp.float32),
                   jax.ShapeDtypeStruct((128, 144), jnp.float32)],
    )(accA, skip, o, p["We"][:, :64], p["We"][:, 64:], p["lin_W"],
      p["lin_b"][None], pt)


def _stats_to_mi(stats, eps=1e-5):
    s1 = stats[:, 0:64].sum(axis=1)
    s2 = stats[:, 64:128].sum(axis=1)
    cnt = stats[:, 128]
    norm = jnp.maximum(cnt, 1.0) * 64.0
    mean = s1 / norm
    var = s2 / norm - mean * mean
    inv = 1.0 / jnp.sqrt(var + eps)
    z = jnp.zeros((128, 126), jnp.float32)
    return jnp.concatenate([mean[:, None], inv[:, None], z], axis=1)


def _t3_body(y_ref, stats_ref, pb_ref, pt_ref, w1, b1, w2, b2,
             y2_ref, stats2_ref):
    i = pl.program_id(0)
    mi = jnp.dot(pb_ref[...], _stats_to_mi(stats_ref[...]),
                 preferred_element_type=jnp.float32)
    o1 = (y_ref[...] - mi[:, 0:1]) * mi[:, 1:2]
    h = _leaky(jnp.dot(o1, w1[...], preferred_element_type=jnp.float32) + b1[...])
    y2 = o1 + jnp.dot(h, w2[...], preferred_element_type=jnp.float32) + b2[...]
    y2_ref[...] = y2
    yy = jnp.concatenate([y2, y2 * y2, jnp.ones((y2.shape[0], 16), jnp.float32)], axis=1)
    st = jnp.dot(pt_ref[...], yy, preferred_element_type=jnp.float32)

    @pl.when(i == 0)
    def _():
        stats2_ref[...] = jnp.zeros_like(stats2_ref)

    stats2_ref[...] += st


def _t3(y, stats, pmat, pt, p):
    return pl.pallas_call(
        _t3_body,
        grid=(NPAD // BM,),
        in_specs=[_rows(64), _full((128, 144)), _rows(128),
                  pl.BlockSpec((128, BM), lambda i: (0, i)),
                  _full((64, 256)), _full((1, 256)), _full((256, 64)),
                  _full((1, 64))],
        out_specs=[_rows(64), _full((128, 144))],
        out_shape=[jax.ShapeDtypeStruct((NPAD, 64), jnp.float32),
                   jax.ShapeDtypeStruct((128, 144), jnp.float32)],
    )(y, stats, pmat, pt, p["ff"]["Ws"][0], p["ff"]["bs"][0][None],
      p["ff"]["Ws"][1], p["ff"]["bs"][1][None])


def _t4_body(y2_ref, stats_ref, pb_ref, o_ref):
    mi = jnp.dot(pb_ref[...], _stats_to_mi(stats_ref[...]),
                 preferred_element_type=jnp.float32)
    o_ref[...] = (y2_ref[...] - mi[:, 0:1]) * mi[:, 1:2]


def _t4(y2, stats2, pmat):
    return pl.pallas_call(
        _t4_body,
        grid=(NPAD // BM,),
        in_specs=[_rows(64), _full((128, 144)), _rows(128)],
        out_specs=_rows(64),
        out_shape=jax.ShapeDtypeStruct((NPAD, 64), jnp.float32),
    )(y2, stats2, pmat)


def _tfa_body(o_ref, cr_ref, pt_ref, m_ref):
    i = pl.program_id(0)
    yy = jnp.concatenate([o_ref[...], cr_ref[...]], axis=1)
    st = jnp.dot(pt_ref[...], yy, preferred_element_type=jnp.float32)

    @pl.when(i == 0)
    def _():
        m_ref[...] = jnp.zeros_like(m_ref)

    m_ref[...] += st


def _tfa(o, colreal, pt):
    return pl.pallas_call(
        _tfa_body,
        grid=(NPAD // BM,),
        in_specs=[_rows(64), _rows(16), pl.BlockSpec((128, BM), lambda i: (0, i))],
        out_specs=_full((128, 80)),
        out_shape=jax.ShapeDtypeStruct((128, 80), jnp.float32),
    )(o, colreal, pt)


def _tfb_body(m_ref, ov_ref, c_ref, glob_ref):
    m = m_ref[...]
    ov = ov_ref[...]
    gmp = (m[:, 0:64] - ov) / jnp.maximum(m[:, 64:65], 1.0)
    glob_ref[...] = jnp.concatenate([gmp, ov, c_ref[...]], axis=1)


def _tfb(m, o_virt, c):
    return pl.pallas_call(
        _tfb_body,
        grid=(1,),
        in_specs=[_full((128, 80)), _full((128, 64)), _full((128, 64))],
        out_specs=_full((128, 192)),
        out_shape=jax.ShapeDtypeStruct((128, 192), jnp.float32),
    )(m, o_virt, c)


def _tfc_body(o_ref, pb_ref, c_ref, out_ref):
    cb = jnp.dot(pb_ref[...], c_ref[...], preferred_element_type=jnp.float32)
    out_ref[...] = jnp.concatenate([o_ref[...], cb], axis=1)


def _tfc(o, pmat, c, n):
    bm = 1000
    return pl.pallas_call(
        _tfc_body,
        grid=(n // bm,),
        in_specs=[pl.BlockSpec((bm, 64), lambda i: (i, 0)),
                  pl.BlockSpec((bm, 128), lambda i: (i, 0)),
                  _full((128, 64))],
        out_specs=pl.BlockSpec((bm, 128), lambda i: (i, 0)),
        out_shape=jax.ShapeDtypeStruct((n, 128), jnp.float32),
    )(o, pmat, c)


def _ldiv_body(la_ref, out_ref):
    la = la_ref[...]
    out_ref[...] = la[:, 0:64] / jnp.maximum(la[:, 64:65], 1.0)


def _ldiv(la):
    return pl.pallas_call(
        _ldiv_body,
        grid=(NPAD // BM,),
        in_specs=[_rows(80)],
        out_specs=_rows(64),
        out_shape=jax.ShapeDtypeStruct((NPAD, 64), jnp.float32),
    )(la)


def kernel(x, edge_index, edge_attr, batch, cond, params):
    N, G = x.shape[0], cond.shape[0]
    o = _mlp3(x, params["x2h"])
    e = _mlp3(edge_attr, params["e2h"])
    c = _mlp3(cond, params["c2h"], bm=128)

    u = jnp.arange(N, dtype=edge_index.dtype)
    v = batch.astype(edge_index.dtype) + N
    n_total = N + G
    NE = edge_index.shape[1]
    o = jnp.concatenate([o, c,
                         jnp.zeros((NPAD - n_total, 64), jnp.float32)], axis=0)

    rn = jnp.arange(NPAD)
    batch_pad = jnp.concatenate([
        batch.astype(jnp.int32), jnp.arange(G, dtype=jnp.int32),
        jnp.zeros((NPAD - n_total,), jnp.int32)])
    pmat = ((batch_pad[:, None] == jnp.arange(G, dtype=jnp.int32)[None, :])
            & (rn[:, None] < n_total)).astype(jnp.float32)
    pt = pmat.T
    colreal = ((rn[:, None] < N)
               & (jnp.arange(16)[None, :] == 0)).astype(jnp.float32)

    ar_n = jnp.arange(N, dtype=jnp.int32)
    new_of_old = jnp.concatenate([ar_n + 1 + ar_n // 79,
                                  80 * jnp.arange(G, dtype=jnp.int32)])
    arr = jnp.full((NPAD,), -1, jnp.int32).at[new_of_old].set(
        jnp.arange(n_total, dtype=jnp.int32))
    unused = arr < 0
    old_of_new = jnp.where(unused,
                           n_total + jnp.cumsum(unused.astype(jnp.int32)) - 1,
                           arr)
    oon2d = old_of_new.reshape(NPAD // 32, 32)

    def _sortset(srcs, dsts, eids, grans):
        E = dsts.shape[0]
        E_pad = 128 * pl.cdiv(E, 128) + EDGE_SLACK
        pad_e = E_pad - E
        d32 = dsts.astype(jnp.int32)
        dstsN = jnp.where(d32 < N_NODES, d32 + 1 + d32 // 79,
                          80 * (d32 - N_NODES))
        perm = jnp.argsort(dstsN)

        def _padi(a):
            return jnp.concatenate([a.astype(jnp.int32),
                                    jnp.zeros((pad_e,), jnp.int32)])

        dst_sorted = dstsN[perm]
        offlist = []
        for gran in grans:
            nb = NPAD // gran
            offs = jnp.searchsorted(dst_sorted,
                                    jnp.arange(nb + 1, dtype=jnp.int32) * gran)
            npadofs = 16 * pl.cdiv(nb + 16, 16)
            offlist.append(jnp.concatenate([
                offs.astype(jnp.int32),
                jnp.full((npadofs - nb - 1,), E, jnp.int32)]))
        src_p = _padi(srcs[perm]) if srcs is not None else None
        return src_p, _padi(dst_sorted), _padi(eids[perm]), offlist

    src0 = jnp.concatenate([edge_index[0], u, v])
    dst0 = jnp.concatenate([edge_index[1], v, u])
    eid0 = jnp.concatenate([jnp.arange(NE, dtype=jnp.int32),
                            jnp.full((2 * N,), NE, jnp.int32)])
    _, dst0_s, eid0_s, (offs0,) = _sortset(None, dst0, eid0, [ROWS_W])

    e_p_row = jnp.zeros((1, 64), jnp.float32).at[0, 0].set(1.0)
    e_store0 = jnp.concatenate([e, e_p_row,
                                jnp.zeros((n_total + 7, 64), jnp.float32)])
    la = _sc_loop_attr(e_store0, dst0_s, eid0_s, offs0, oon2d)
    loop_attr = _ldiv(la)
    e_store = lax.dynamic_update_slice(e_store0, loop_attr[:n_total], (NE + 1, 0))

    sl = jnp.arange(n_total, dtype=edge_index.dtype)
    aug_src = jnp.concatenate([src0, sl])
    aug_dst = jnp.concatenate([dst0, sl])
    eid = jnp.concatenate([eid0, NE + 1 + jnp.arange(n_total, dtype=jnp.int32)])
    src_s, dst_s, eid_s, (offs, offs64) = _sortset(aug_src, aug_dst, eid,
                                                   [ROWS_W, HALF_W])

    for p in params["layers"]:
        agg = _sc_gen_msg(o, e_store, src_s, dst_s, eid_s, offs, oon2d)
        dtab, stab, skip = _t1(o, agg, loop_attr, p)
        accA = _sc_attn(dtab, stab, e_store, src_s, dst_s, eid_s, offs64, oon2d)
        y, stats = _t2(accA, skip, o, pt, p)
        y2, stats2 = _t3(y, stats, pmat, pt, p)
        o = _t4(y2, stats2, pmat)

    m = _tfa(o, colreal, pt)
    o_virt = o[N:N + G]
    glob = _tfb(m, o_virt, c)
    o_final = _tfc(o, pmat, c, N)
    return (o_final, glob)


# speedup vs baseline: 1.3667x; 1.0359x over previous
import math
from functools import partial

import jax
import jax.numpy as jnp
from jax import lax
from jax.experimental import pallas as pl
from jax.experimental.pallas import tpu as pltpu
from jax.experimental.pallas import tpu_sc as plsc

N_NODES = 10000
N_GRAPHS = 128
N_EDGES_IN = 160000
NUM_EMB = 64
NUM_HEADS = 2

NC, NS, LANES = 2, 16, 16
NW = NC * NS
NPAD = 10240
ROWS_W = NPAD // NW
EDGE_SLACK = 1280


def _sc_mesh():
    return plsc.VectorSubcoreMesh(core_axis_name="c", subcore_axis_name="s")


def _sc_params():
    return pltpu.CompilerParams(use_tc_tiling_on_sc=False,
                                needs_layout_passes=False)


def _zero_acc(acc, rows, width):
    z = jnp.zeros((LANES,), jnp.float32)

    @pl.loop(0, rows)
    def _(r):
        for f in range(width // LANES):
            acc[r, pl.ds(f * LANES, LANES)] = z


def _pipeline(ch, lo, hi, idx_streams, gath_streams, isems, gsems, compute):
    lo8 = (lo // 8) * 8
    nj = (hi - lo8 + ch - 1) // ch
    npair = (nj + 1) // 2

    def fire_idx(p, base):
        for arr, buf in idx_streams:
            pltpu.make_async_copy(arr.at[pl.ds(base, ch)], buf.at[p], isems[p]).start()

    def wait_idx(p):
        for arr, buf in idx_streams:
            pltpu.make_async_copy(arr.at[pl.ds(0, ch)], buf.at[p], isems[p]).wait()

    def fire_gath(p):
        for tab, ibuf, dbuf in gath_streams:
            pltpu.make_async_copy(tab.at[ibuf.at[p]], dbuf.at[p], gsems[p]).start()

    def wait_gath(p):
        for tab, ibuf, dbuf in gath_streams:
            pltpu.make_async_copy(tab.at[ibuf.at[p]], dbuf.at[p], gsems[p]).wait()

    fire_idx(0, lo8)
    wait_idx(0)
    fire_gath(0)
    fire_idx(1, lo8 + ch)

    @pl.loop(0, npair)
    def _(jj):
        b0 = lo8 + (2 * jj) * ch
        for p in (0, 1):
            base = b0 + p * ch
            wait_idx(1 - p)
            fire_gath(1 - p)
            wait_gath(p)
            compute(p, base, lo, hi)
            fire_idx(p, base + 2 * ch)

    wait_gath(0)
    wait_idx(1)


def _flush_scatter(acc, out_hbm, oon2d, fidx, fsem, base32, n32):
    pltpu.sync_copy(oon2d.at[pl.ds(base32, n32)], fidx)
    cps = [pltpu.make_async_copy(acc.at[pl.ds(t * 32, 32)],
                                 out_hbm.at[fidx.at[t]], fsem)
           for t in range(n32)]
    for cp in cps:
        cp.start()
    for cp in cps:
        cp.wait()


GEN_CH = 256


def _gen_msg_body(o_hbm, es_hbm, src_hbm, dst_hbm, eid_hbm, off_hbm, oon_hbm,
                  out_hbm, sidx, didx, eidx, off_v, fidx, gbuf, ebuf, acc,
                  si0, si1, sg0, sg1, fsem):
    c = lax.axis_index("c")
    s = lax.axis_index("s")
    w = c * NS + s
    row0 = w * ROWS_W

    _zero_acc(acc, ROWS_W, 64)
    pltpu.sync_copy(off_hbm, off_v)
    iota = lax.iota(jnp.int32, LANES)
    ovec = plsc.load_gather(off_v, [w + iota])

    def compute(p, base, lo, hi):
        @pl.loop(0, GEN_CH // LANES)
        def _(g):
            dvec = jnp.clip(didx[p, pl.ds(g * LANES, LANES)] - row0, 0, ROWS_W - 1)
            for i in range(LANES):
                r = g * LANES + i
                pos = base + r
                valid = jnp.logical_and(pos >= lo, pos < hi)
                m = jnp.broadcast_to(valid, (LANES,))
                rowv = jnp.broadcast_to(dvec[i], (LANES,))
                for f in range(4):
                    sl = pl.ds(f * LANES, LANES)
                    vv = jnp.maximum(gbuf[p, r, sl] + ebuf[p, r, sl], 0.0) + 1e-7
                    plsc.addupdate_scatter(acc, [rowv, iota + f * LANES], vv, mask=m)

    _pipeline(GEN_CH, ovec[0], ovec[1],
              [(src_hbm, sidx), (dst_hbm, didx), (eid_hbm, eidx)],
              [(o_hbm, sidx, gbuf), (es_hbm, eidx, ebuf)],
              (si0, si1), (sg0, sg1), compute)

    _flush_scatter(acc, out_hbm, oon_hbm, fidx, fsem, w * 10, 10)


@jax.jit
def _sc_gen_msg(o_pad, e_store, src_s, dst_s, eid_s, offs, oon2d):
    k = pl.kernel(
        _gen_msg_body,
        out_type=jax.ShapeDtypeStruct((NPAD, 64), jnp.float32),
        mesh=_sc_mesh(),
        compiler_params=_sc_params(),
        scratch_types=[
            pltpu.VMEM((2, GEN_CH), jnp.int32),
            pltpu.VMEM((2, GEN_CH), jnp.int32),
            pltpu.VMEM((2, GEN_CH), jnp.int32),
            pltpu.VMEM((48,), jnp.int32),
            pltpu.VMEM((10, 32), jnp.int32),
            pltpu.VMEM((2, GEN_CH, 64), jnp.float32),
            pltpu.VMEM((2, GEN_CH, 64), jnp.float32),
            pltpu.VMEM((ROWS_W, 64), jnp.float32),
            pltpu.SemaphoreType.DMA,
            pltpu.SemaphoreType.DMA,
            pltpu.SemaphoreType.DMA,
            pltpu.SemaphoreType.DMA,
            pltpu.SemaphoreType.DMA,
        ],
    )
    return k(o_pad, e_store, src_s, dst_s, eid_s, offs, oon2d)


BC_CH = 64
HALF_W = ROWS_W // 2


def _attn_body(dtab, stab, es_hbm, src_hbm, dst_hbm, eid_hbm, off_hbm, oon_hbm,
               out_hbm, sidx, didx, eidx, off_v, fidx, dgath, sgath, ebuf, acc,
               si0, si1, sg0, sg1, fsem):
    c = lax.axis_index("c")
    s = lax.axis_index("s")
    w = c * NS + s

    pltpu.sync_copy(off_hbm, off_v)
    iota = lax.iota(jnp.int32, LANES)

    for half in range(2):
        row0 = w * ROWS_W + half * HALF_W
        _zero_acc(acc, HALF_W, 272)
        ovec = plsc.load_gather(off_v, [2 * w + half + iota])

        def compute(p, base, lo, hi):
            @pl.loop(0, BC_CH // LANES)
            def _(g):
                dvec = jnp.clip(didx[p, pl.ds(g * LANES, LANES)] - row0, 0, HALF_W - 1)
                for i in range(LANES):
                    r = g * LANES + i
                    pos = base + r
                    valid = jnp.logical_and(pos >= lo, pos < hi)
                    m = jnp.broadcast_to(valid, (LANES,))
                    rowv = jnp.broadcast_to(dvec[i], (LANES,))
                    qk = [dgath[p, r, pl.ds(f * LANES, LANES)] for f in range(8)]
                    qe = [dgath[p, r, pl.ds(128 + f * LANES, LANES)] for f in range(8)]
                    kv = [sgath[p, r, pl.ds(f * LANES, LANES)] for f in range(8)]
                    vv = [sgath[p, r, pl.ds(128 + f * LANES, LANES)] for f in range(8)]
                    ev = [ebuf[p, r, pl.ds(f * LANES, LANES)] for f in range(4)]
                    svec = dgath[p, r, pl.ds(256, LANES)]
                    m0 = qk[0] * kv[0]
                    m1 = qk[4] * kv[4]
                    for f in range(1, 4):
                        m0 = m0 + qk[f] * kv[f]
                        m1 = m1 + qk[4 + f] * kv[4 + f]
                    for f in range(4):
                        m0 = m0 + qe[f] * ev[f]
                        m1 = m1 + qe[4 + f] * ev[f]
                    a0 = jnp.sum(m0) * 0.125 - svec[0]
                    a1 = jnp.sum(m1) * 0.125 - svec[1]
                    zb0 = jnp.exp(jnp.broadcast_to(a0, (LANES,)))
                    zb1 = jnp.exp(jnp.broadcast_to(a1, (LANES,)))
                    for f in range(4):
                        sl = iota + f * LANES
                        plsc.addupdate_scatter(acc, [rowv, sl], vv[f] * zb0, mask=m)
                        plsc.addupdate_scatter(acc, [rowv, sl + 64], vv[4 + f] * zb1, mask=m)
                        plsc.addupdate_scatter(acc, [rowv, sl + 128], ev[f] * zb0, mask=m)
                        plsc.addupdate_scatter(acc, [rowv, sl + 192], ev[f] * zb1, mask=m)
                    zrow = jnp.where(iota == 0, zb0, jnp.where(iota == 1, zb1, 0.0))
                    plsc.addupdate_scatter(acc, [rowv, iota + 256], zrow, mask=m)

        _pipeline(BC_CH, ovec[0], ovec[1],
                  [(src_hbm, sidx), (dst_hbm, didx), (eid_hbm, eidx)],
                  [(dtab, didx, dgath), (stab, sidx, sgath), (es_hbm, eidx, ebuf)],
                  (si0, si1), (sg0, sg1), compute)

        _flush_scatter(acc, out_hbm, oon_hbm, fidx, fsem, w * 10 + half * 5, 5)


@jax.jit
def _sc_attn(dtab, stab, e_store, src_s, dst_s, eid_s, offs64, oon2d):
    k = pl.kernel(
        _attn_body,
        out_type=jax.ShapeDtypeStruct((NPAD, 272), jnp.float32),
        mesh=_sc_mesh(),
        compiler_params=_sc_params(),
        scratch_types=[
            pltpu.VMEM((2, BC_CH), jnp.int32),
            pltpu.VMEM((2, BC_CH), jnp.int32),
            pltpu.VMEM((2, BC_CH), jnp.int32),
            pltpu.VMEM((80,), jnp.int32),
            pltpu.VMEM((5, 32), jnp.int32),
            pltpu.VMEM((2, BC_CH, 272), jnp.float32),
            pltpu.VMEM((2, BC_CH, 256), jnp.float32),
            pltpu.VMEM((2, BC_CH, 64), jnp.float32),
            pltpu.VMEM((HALF_W, 272), jnp.float32),
            pltpu.SemaphoreType.DMA,
            pltpu.SemaphoreType.DMA,
            pltpu.SemaphoreType.DMA,
            pltpu.SemaphoreType.DMA,
            pltpu.SemaphoreType.DMA,
        ],
    )
    return k(dtab, stab, e_store, src_s, dst_s, eid_s, offs64, oon2d)


LA_CH = 256


def _lattr_body(es_hbm, dst_hbm, eid_hbm, off_hbm, oon_hbm, out_hbm,
                didx, eidx, off_v, fidx, ebuf, acc, si0, si1, sg0, sg1, fsem):
    c = lax.axis_index("c")
    s = lax.axis_index("s")
    w = c * NS + s
    row0 = w * ROWS_W

    _zero_acc(acc, ROWS_W, 80)
    pltpu.sync_copy(off_hbm, off_v)
    iota = lax.iota(jnp.int32, LANES)
    ovec = plsc.load_gather(off_v, [w + iota])
    onerow = jnp.where(iota == 0, 1.0, 0.0)

    def compute(p, base, lo, hi):
        @pl.loop(0, LA_CH // LANES)
        def _(g):
            dvec = jnp.clip(didx[p, pl.ds(g * LANES, LANES)] - row0, 0, ROWS_W - 1)
            evec = eidx[p, pl.ds(g * LANES, LANES)]
            for i in range(LANES):
                r = g * LANES + i
                pos = base + r
                valid = jnp.logical_and(jnp.logical_and(pos >= lo, pos < hi),
                                        evec[i] <= N_EDGES_IN)
                m = jnp.broadcast_to(valid, (LANES,))
                rowv = jnp.broadcast_to(dvec[i], (LANES,))
                for f in range(4):
                    plsc.addupdate_scatter(acc, [rowv, iota + f * LANES],
                                           ebuf[p, r, pl.ds(f * LANES, LANES)], mask=m)
                plsc.addupdate_scatter(acc, [rowv, iota + 64], onerow, mask=m)

    _pipeline(LA_CH, ovec[0], ovec[1],
              [(dst_hbm, didx), (eid_hbm, eidx)],
              [(es_hbm, eidx, ebuf)],
              (si0, si1), (sg0, sg1), compute)

    _flush_scatter(acc, out_hbm, oon_hbm, fidx, fsem, w * 10, 10)


@jax.jit
def _sc_loop_attr(e_store, dst_s, eid_s, offs, oon2d):
    k = pl.kernel(
        _lattr_body,
        out_type=jax.ShapeDtypeStruct((NPAD, 80), jnp.float32),
        mesh=_sc_mesh(),
        compiler_params=_sc_params(),
        scratch_types=[
            pltpu.VMEM((2, LA_CH), jnp.int32),
            pltpu.VMEM((2, LA_CH), jnp.int32),
            pltpu.VMEM((48,), jnp.int32),
            pltpu.VMEM((10, 32), jnp.int32),
            pltpu.VMEM((2, LA_CH, 64), jnp.float32),
            pltpu.VMEM((ROWS_W, 80), jnp.float32),
            pltpu.SemaphoreType.DMA,
            pltpu.SemaphoreType.DMA,
            pltpu.SemaphoreType.DMA,
            pltpu.SemaphoreType.DMA,
            pltpu.SemaphoreType.DMA,
        ],
    )
    return k(e_store, dst_s, eid_s, offs, oon2d)


def _leaky(x):
    return jnp.where(x >= 0, x, 0.01 * x)


def _mlp3_body(a_ref, w1, b1, w2, b2, w3, b3, o_ref):
    a = a_ref[...]
    h = _leaky(jnp.dot(a, w1[...], preferred_element_type=jnp.float32) + b1[...])
    h = _leaky(jnp.dot(h, w2[...], preferred_element_type=jnp.float32) + b2[...])
    h = jnp.dot(h, w3[...], preferred_element_type=jnp.float32) + b3[...]
    o_ref[...] = h


def _mlp3(a, p, bm=2048):
    M, F = a.shape
    D = p["Ws"][2].shape[1]
    grid = (pl.cdiv(M, bm),)
    full = lambda shape: pl.BlockSpec(shape, lambda i: (0,) * len(shape))
    return pl.pallas_call(
        _mlp3_body,
        grid=grid,
        in_specs=[
            pl.BlockSpec((bm, F), lambda i: (i, 0)),
            full(p["Ws"][0].shape), full((1, p["bs"][0].shape[0])),
            full(p["Ws"][1].shape), full((1, p["bs"][1].shape[0])),
            full(p["Ws"][2].shape), full((1, p["bs"][2].shape[0])),
        ],
        out_specs=pl.BlockSpec((bm, D), lambda i: (i, 0)),
        out_shape=jax.ShapeDtypeStruct((M, D), jnp.float32),
    )(a, p["Ws"][0], p["bs"][0][None], p["Ws"][1], p["bs"][1][None],
      p["Ws"][2], p["bs"][2][None])


BM = 1024


def _full(shape):
    return pl.BlockSpec(shape, lambda i: (0,) * len(shape))


def _rows(width):
    return pl.BlockSpec((BM, width), lambda i: (i, 0))


def _t1_body(o_ref, agg_ref, la_ref, genW, genb, wqkvs, bqkvs, weT2, we,
             dtab_ref, stab_ref, skip_ref):
    o = o_ref[...]
    genout = jnp.dot(agg_ref[...] + o, genW[...],
                     preferred_element_type=jnp.float32) + genb[...]
    xc = jnp.concatenate([o, genout], axis=1)
    qkvs = jnp.dot(xc, wqkvs[...], preferred_element_type=jnp.float32) + bqkvs[...]
    q = qkvs[:, 0:128]
    k = qkvs[:, 128:256]
    vv = qkvs[:, 256:384]
    skip_ref[...] = qkvs[:, 384:512]
    qWe = jnp.dot(q, weT2[...], preferred_element_type=jnp.float32)
    elC = jnp.dot(la_ref[...], we[...], preferred_element_type=jnp.float32)
    kc = k + elC
    s0 = (q[:, :64] * kc[:, :64]).sum(axis=1, keepdims=True) * 0.125
    s1 = (q[:, 64:] * kc[:, 64:]).sum(axis=1, keepdims=True) * 0.125
    dtab_ref[...] = jnp.concatenate(
        [q, qWe, s0, s1, jnp.zeros((o.shape[0], 14), jnp.float32)], axis=1)
    stab_ref[...] = jnp.concatenate([k, vv], axis=1)


def _t1(o, agg, la_pad, p):
    wqkvs = jnp.concatenate([p["Wq"], p["Wk"], p["Wv"], p["Wsk"]], axis=1)
    bqkvs = jnp.concatenate([p["bq"], p["bk"], p["bv"], p["bsk"]])[None]
    z64 = jnp.zeros((64, 64), jnp.float32)
    weT2 = jnp.concatenate([
        jnp.concatenate([p["We"][:, :64].T, z64], axis=1),
        jnp.concatenate([z64, p["We"][:, 64:].T], axis=1)], axis=0)
    return pl.pallas_call(
        _t1_body,
        grid=(NPAD // BM,),
        in_specs=[_rows(64), _rows(64), _rows(64),
                  _full((64, 64)), _full((1, 64)), _full((128, 512)),
                  _full((1, 512)), _full((128, 128)), _full((64, 128))],
        out_specs=[_rows(272), _rows(256), _rows(128)],
        out_shape=[jax.ShapeDtypeStruct((NPAD, 272), jnp.float32),
                   jax.ShapeDtypeStruct((NPAD, 256), jnp.float32),
                   jax.ShapeDtypeStruct((NPAD, 128), jnp.float32)],
    )(o, agg, la_pad, p["gen_W"], p["gen_b"][None], wqkvs, bqkvs, weT2, p["We"])


def _t2_body(acc_ref, skip_ref, o_ref, we0, we1, linW, linb, pt_ref,
             y_ref, stats_ref):
    i = pl.program_id(0)
    a = acc_ref[...]
    out0 = (a[:, 0:64] + jnp.dot(a[:, 128:192], we0[...],
                                 preferred_element_type=jnp.float32)) / a[:, 256:257]
    out1 = (a[:, 64:128] + jnp.dot(a[:, 192:256], we1[...],
                                   preferred_element_type=jnp.float32)) / a[:, 257:258]
    t = jnp.concatenate([out0, out1], axis=1) + skip_ref[...]
    y = o_ref[...] + jnp.dot(t, linW[...], preferred_element_type=jnp.float32) + linb[...]
    y_ref[...] = y
    yy = jnp.concatenate([y, y * y, jnp.ones((y.shape[0], 16), jnp.float32)], axis=1)
    st = jnp.dot(pt_ref[...], yy, preferred_element_type=jnp.float32)

    @pl.when(i == 0)
    def _():
        stats_ref[...] = jnp.zeros_like(stats_ref)

    stats_ref[...] += st


def _t2(accA, skip, o, pt, p):
    return pl.pallas_call(
        _t2_body,
        grid=(NPAD // BM,),
        in_specs=[_rows(272), _rows(128), _rows(64),
                  _full((64, 64)), _full((64, 64)), _full((128, 64)),
                  _full((1, 64)), pl.BlockSpec((128, BM), lambda i: (0, i))],
        out_specs=[_rows(64), _full((128, 144))],
        out_shape=[jax.ShapeDtypeStruct((NPAD, 64), jnp.float32),
                   jax.ShapeDtypeStruct((128, 144), jnp.float32)],
    )(accA, skip, o, p["We"][:, :64], p["We"][:, 64:], p["lin_W"],
      p["lin_b"][None], pt)


def _stats_to_mi(stats, eps=1e-5):
    s1 = stats[:, 0:64].sum(axis=1)
    s2 = stats[:, 64:128].sum(axis=1)
    cnt = stats[:, 128]
    norm = jnp.maximum(cnt, 1.0) * 64.0
    mean = s1 / norm
    var = s2 / norm - mean * mean
    inv = 1.0 / jnp.sqrt(var + eps)
    z = jnp.zeros((128, 126), jnp.float32)
    return jnp.concatenate([mean[:, None], inv[:, None], z], axis=1)


def _t3_body(y_ref, stats_ref, pb_ref, pt_ref, w1, b1, w2, b2,
             y2_ref, stats2_ref):
    i = pl.program_id(0)
    mi = jnp.dot(pb_ref[...], _stats_to_mi(stats_ref[...]),
                 preferred_element_type=jnp.float32)
    o1 = (y_ref[...] - mi[:, 0:1]) * mi[:, 1:2]
    h = _leaky(jnp.dot(o1, w1[...], preferred_element_type=jnp.float32) + b1[...])
    y2 = o1 + jnp.dot(h, w2[...], preferred_element_type=jnp.float32) + b2[...]
    y2_ref[...] = y2
    yy = jnp.concatenate([y2, y2 * y2, jnp.ones((y2.shape[0], 16), jnp.float32)], axis=1)
    st = jnp.dot(pt_ref[...], yy, preferred_element_type=jnp.float32)

    @pl.when(i == 0)
    def _():
        stats2_ref[...] = jnp.zeros_like(stats2_ref)

    stats2_ref[...] += st


def _t3(y, stats, pmat, pt, p):
    return pl.pallas_call(
        _t3_body,
        grid=(NPAD // BM,),
        in_specs=[_rows(64), _full((128, 144)), _rows(128),
                  pl.BlockSpec((128, BM), lambda i: (0, i)),
                  _full((64, 256)), _full((1, 256)), _full((256, 64)),
                  _full((1, 64))],
        out_specs=[_rows(64), _full((128, 144))],
        out_shape=[jax.ShapeDtypeStruct((NPAD, 64), jnp.float32),
                   jax.ShapeDtypeStruct((128, 144), jnp.float32)],
    )(y, stats, pmat, pt, p["ff"]["Ws"][0], p["ff"]["bs"][0][None],
      p["ff"]["Ws"][1], p["ff"]["bs"][1][None])


def _t4_body(y2_ref, stats_ref, pb_ref, o_ref):
    mi = jnp.dot(pb_ref[...], _stats_to_mi(stats_ref[...]),
                 preferred_element_type=jnp.float32)
    o_ref[...] = (y2_ref[...] - mi[:, 0:1]) * mi[:, 1:2]


def _t4(y2, stats2, pmat):
    return pl.pallas_call(
        _t4_body,
        grid=(NPAD // BM,),
        in_specs=[_rows(64), _full((128, 144)), _rows(128)],
        out_specs=_rows(64),
        out_shape=jax.ShapeDtypeStruct((NPAD, 64), jnp.float32),
    )(y2, stats2, pmat)


def _tfa_body(o_ref, cr_ref, pt_ref, m_ref):
    i = pl.program_id(0)
    yy = jnp.concatenate([o_ref[...], cr_ref[...]], axis=1)
    st = jnp.dot(pt_ref[...], yy, preferred_element_type=jnp.float32)

    @pl.when(i == 0)
    def _():
        m_ref[...] = jnp.zeros_like(m_ref)

    m_ref[...] += st


def _tfa(o, colreal, pt):
    return pl.pallas_call(
        _tfa_body,
        grid=(NPAD // BM,),
        in_specs=[_rows(64), _rows(16), pl.BlockSpec((128, BM), lambda i: (0, i))],
        out_specs=_full((128, 80)),
        out_shape=jax.ShapeDtypeStruct((128, 80), jnp.float32),
    )(o, colreal, pt)


def _tfb_body(m_ref, ov_ref, c_ref, glob_ref):
    m = m_ref[...]
    ov = ov_ref[...]
    gmp = (m[:, 0:64] - ov) / jnp.maximum(m[:, 64:65], 1.0)
    glob_ref[...] = jnp.concatenate([gmp, ov, c_ref[...]], axis=1)


def _tfb(m, o_virt, c):
    return pl.pallas_call(
        _tfb_body,
        grid=(1,),
        in_specs=[_full((128, 80)), _full((128, 64)), _full((128, 64))],
        out_specs=_full((128, 192)),
        out_shape=jax.ShapeDtypeStruct((128, 192), jnp.float32),
    )(m, o_virt, c)


def _tfc_body(o_ref, pb_ref, c_ref, out_ref):
    cb = jnp.dot(pb_ref[...], c_ref[...], preferred_element_type=jnp.float32)
    out_ref[...] = jnp.concatenate([o_ref[...], cb], axis=1)


def _tfc(o, pmat, c, n):
    bm = 1000
    return pl.pallas_call(
        _tfc_body,
        grid=(n // bm,),
        in_specs=[pl.BlockSpec((bm, 64), lambda i: (i, 0)),
                  pl.BlockSpec((bm, 128), lambda i: (i, 0)),
                  _full((128, 64))],
        out_specs=pl.BlockSpec((bm, 128), lambda i: (i, 0)),
        out_shape=jax.ShapeDtypeStruct((n, 128), jnp.float32),
    )(o, pmat, c)


def _ldiv_body(la_ref, out_ref):
    la = la_ref[...]
    out_ref[...] = la[:, 0:64] / jnp.maximum(la[:, 64:65], 1.0)


def _ldiv(la):
    return pl.pallas_call(
        _ldiv_body,
        grid=(NPAD // BM,),
        in_specs=[_rows(80)],
        out_specs=_rows(64),
        out_shape=jax.ShapeDtypeStruct((NPAD, 64), jnp.float32),
    )(la)


def kernel(x, edge_index, edge_attr, batch, cond, params):
    N, G = x.shape[0], cond.shape[0]
    o = _mlp3(x, params["x2h"])
    e = _mlp3(edge_attr, params["e2h"])
    c = _mlp3(cond, params["c2h"], bm=128)

    u = jnp.arange(N, dtype=edge_index.dtype)
    v = batch.astype(edge_index.dtype) + N
    n_total = N + G
    NE = edge_index.shape[1]
    o = jnp.concatenate([o, c,
                         jnp.zeros((NPAD - n_total, 64), jnp.float32)], axis=0)

    rn = jnp.arange(NPAD)
    batch_pad = jnp.concatenate([
        batch.astype(jnp.int32), jnp.arange(G, dtype=jnp.int32),
        jnp.zeros((NPAD - n_total,), jnp.int32)])
    pmat = ((batch_pad[:, None] == jnp.arange(G, dtype=jnp.int32)[None, :])
            & (rn[:, None] < n_total)).astype(jnp.float32)
    pt = pmat.T
    colreal = ((rn[:, None] < N)
               & (jnp.arange(16)[None, :] == 0)).astype(jnp.float32)

    ar_n = jnp.arange(N, dtype=jnp.int32)
    new_of_old = jnp.concatenate([ar_n + 1 + ar_n // 79,
                                  80 * jnp.arange(G, dtype=jnp.int32)])
    arr = jnp.full((NPAD,), -1, jnp.int32).at[new_of_old].set(
        jnp.arange(n_total, dtype=jnp.int32))
    unused = arr < 0
    old_of_new = jnp.where(unused,
                           n_total + jnp.cumsum(unused.astype(jnp.int32)) - 1,
                           arr)
    oon2d = old_of_new.reshape(NPAD // 32, 32)

    def _sortset(srcs, dsts, eids, grans):
        E = dsts.shape[0]
        E_pad = 128 * pl.cdiv(E, 128) + EDGE_SLACK
        pad_e = E_pad - E
        d32 = dsts.astype(jnp.int32)
        dstsN = jnp.where(d32 < N_NODES, d32 + 1 + d32 // 79,
                          80 * (d32 - N_NODES))
        perm = jnp.argsort(dstsN)

        def _padi(a):
            return jnp.concatenate([a.astype(jnp.int32),
                                    jnp.zeros((pad_e,), jnp.int32)])

        dst_sorted = dstsN[perm]
        offlist = []
        for gran in grans:
            nb = NPAD // gran
            offs = jnp.searchsorted(dst_sorted,
                                    jnp.arange(nb + 1, dtype=jnp.int32) * gran)
            npadofs = 16 * pl.cdiv(nb + 16, 16)
            offlist.append(jnp.concatenate([
                offs.astype(jnp.int32),
                jnp.full((npadofs - nb - 1,), E, jnp.int32)]))
        src_p = _padi(srcs[perm]) if srcs is not None else None
        return src_p, _padi(dst_sorted), _padi(eids[perm]), offlist

    src0 = jnp.concatenate([edge_index[0], u, v])
    dst0 = jnp.concatenate([edge_index[1], v, u])
    eid0 = jnp.concatenate([jnp.arange(NE, dtype=jnp.int32),
                            jnp.full((2 * N,), NE, jnp.int32)])
    sl = jnp.arange(n_total, dtype=edge_index.dtype)
    aug_src = jnp.concatenate([src0, sl])
    aug_dst = jnp.concatenate([dst0, sl])
    eid = jnp.concatenate([eid0, NE + 1 + jnp.arange(n_total, dtype=jnp.int32)])
    src_s, dst_s, eid_s, (offs, offs64) = _sortset(aug_src, aug_dst, eid,
                                                   [ROWS_W, HALF_W])

    e_p_row = jnp.zeros((1, 64), jnp.float32).at[0, 0].set(1.0)
    e_store0 = jnp.concatenate([e, e_p_row,
                                jnp.zeros((n_total + 7, 64), jnp.float32)])
    la = _sc_loop_attr(e_store0, dst_s, eid_s, offs, oon2d)
    loop_attr = _ldiv(la)
    e_store = lax.dynamic_update_slice(e_store0, loop_attr[:n_total], (NE + 1, 0))

    for p in params["layers"]:
        agg = _sc_gen_msg(o, e_store, src_s, dst_s, eid_s, offs, oon2d)
        dtab, stab, skip = _t1(o, agg, loop_attr, p)
        accA = _sc_attn(dtab, stab, e_store, src_s, dst_s, eid_s, offs64, oon2d)
        y, stats = _t2(accA, skip, o, pt, p)
        y2, stats2 = _t3(y, stats, pmat, pt, p)
        o = _t4(y2, stats2, pmat)

    m = _tfa(o, colreal, pt)
    o_virt = o[N:N + G]
    glob = _tfb(m, o_virt, c)
    o_final = _tfc(o, pmat, c, N)
    return (o_final, glob)
